# baseline (speedup 1.0000x reference)
import functools

import jax
import jax.numpy as jnp
from jax import lax
from jax.experimental import pallas as pl
from jax.experimental.pallas import tpu as pltpu
from jax.experimental.pallas import tpu_sc as plsc

N = 10000
E = 320000
D_IN = 128
H = 128
C = 64

NC = 2
NS = 16
NW = NC * NS
E_PER = E // NW
EB = 128
NCHUNK = E_PER // EB
ET = E_PER - NCHUNK * EB
RB = 624
REM = N - RB * NS
ZR = 208
NP = (NCHUNK - 2) // 2

_mesh = plsc.VectorSubcoreMesh(core_axis_name="c", subcore_axis_name="s")


def _fill_zeros(ref, rows, width):
    def body(i, _):
        def inner(j, _):
            ref[i, pl.ds(j * 16, 16)] = jnp.zeros((16,), jnp.float32)
            return 0
        return lax.fori_loop(0, width // 16, inner, 0)
    lax.fori_loop(0, rows, body, 0)


def _zero_acc(zero_v, acc, sid):
    r0 = sid * RB
    for k in range(RB // ZR):
        pltpu.sync_copy(zero_v, acc.at[pl.ds(r0 + k * ZR, ZR)])

    @pl.when(sid == NS - 1)
    def _zero_tail():
        pltpu.sync_copy(zero_v.at[pl.ds(0, REM)], acc.at[pl.ds(RB * NS, REM)])


def _writeback(acc, out_hbm, cid, sid):
    r0 = sid * RB
    pltpu.sync_copy(acc.at[pl.ds(r0, RB)], out_hbm.at[cid, pl.ds(r0, RB)])

    @pl.when(sid == NS - 1)
    def _write_tail():
        pltpu.sync_copy(acc.at[pl.ds(RB * NS, REM)],
                        out_hbm.at[cid, pl.ds(RB * NS, REM)])


def _deg_body(src_hbm, dst_hbm, out_hbm, in_hbm, i0, i1, it, ones_v, zero_v,
              acc, s0, s1):
    cid = lax.axis_index("c")
    sid = lax.axis_index("s")
    wid = cid * NS + sid
    ibufs = (i0, i1)
    sems = (s0, s1)

    _fill_zeros(zero_v, ZR, H)

    def fill_ones(i, _):
        def inner(j, _):
            ones_v[i, pl.ds(j * 16, 16)] = jnp.ones((16,), jnp.float32)
            return 0
        return lax.fori_loop(0, H // 16, inner, 0)
    lax.fori_loop(0, EB, fill_ones, 0)

    for idx_hbm, o_hbm in ((src_hbm, out_hbm), (dst_hbm, in_hbm)):
        _zero_acc(zero_v, acc, sid)
        plsc.subcore_barrier()

        def start_idx(b, g):
            base = wid * E_PER + g * EB
            pltpu.async_copy(idx_hbm.at[pl.ds(base, EB)], ibufs[b], sems[b])

        def start_scatter(b):
            pltpu.make_async_copy(idx_hbm.at[pl.ds(0, EB)], ibufs[b],
                                  sems[b]).wait()
            pltpu.async_copy(ones_v, acc.at[ibufs[b]], sems[b], add=True)

        def wait_scatter(b):
            pltpu.make_async_copy(ones_v, acc.at[ibufs[b]], sems[b]).wait()

        start_idx(0, 0)

        def body(p, _):
            start_scatter(0)
            start_idx(1, 2 * p + 1)
            wait_scatter(0)
            start_scatter(1)
            start_idx(0, 2 * p + 2)
            wait_scatter(1)
            return 0
        lax.fori_loop(0, NP, body, 0)

        start_scatter(0)
        start_idx(1, NCHUNK - 1)
        wait_scatter(0)
        start_scatter(1)
        wait_scatter(1)

        pltpu.sync_copy(idx_hbm.at[pl.ds(wid * E_PER + NCHUNK * EB, ET)], it)
        pltpu.sync_copy(ones_v.at[pl.ds(0, ET)], acc.at[it], add=True)
        plsc.subcore_barrier()

        _writeback(acc, o_hbm, cid, sid)
        plsc.subcore_barrier()


_deg_kernel = pl.kernel(
    _deg_body,
    out_type=[jax.ShapeDtypeStruct((NC, N, H), jnp.float32),
              jax.ShapeDtypeStruct((NC, N, H), jnp.float32)],
    mesh=_mesh,
    scratch_types=(
        [pltpu.VMEM((EB,), jnp.int32)] * 2
        + [pltpu.VMEM((ET,), jnp.int32),
           pltpu.VMEM((EB, H), jnp.float32),
           pltpu.VMEM((ZR, H), jnp.float32),
           pltpu.VMEM_SHARED((N, H), jnp.float32)]
        + [pltpu.SemaphoreType.DMA] * 2
    ),
)


def _agg_body(h_hbm, src_hbm, dst_hbm, out_hbm, i0, i1, it, r0, r1, rt, acc,
              s0, s1, *, d):
    cid = lax.axis_index("c")
    sid = lax.axis_index("s")
    wid = cid * NS + sid
    ibufs = (i0, i1)
    rbufs = (r0, r1)
    sems = (s0, s1)

    _fill_zeros(r0, EB, d)
    zb = sid * RB
    for k in range(RB // EB):
        pltpu.sync_copy(r0, acc.at[pl.ds(zb + k * EB, EB)])
    pltpu.sync_copy(r0.at[pl.ds(0, RB % EB)],
                    acc.at[pl.ds(zb + RB - RB % EB, RB % EB)])

    @pl.when(sid == NS - 1)
    def _zero_tail():
        pltpu.sync_copy(r0.at[pl.ds(0, REM)], acc.at[pl.ds(RB * NS, REM)])
    plsc.subcore_barrier()

    def start_idx(b, g):
        base = wid * E_PER + g * EB
        pltpu.async_copy(src_hbm.at[pl.ds(base, EB)], ibufs[b].at[0], sems[b])
        pltpu.async_copy(dst_hbm.at[pl.ds(base, EB)], ibufs[b].at[1], sems[b])

    def start_gather(b):
        pltpu.make_async_copy(src_hbm.at[pl.ds(0, EB)], ibufs[b].at[0],
                              sems[b]).wait()
        pltpu.make_async_copy(src_hbm.at[pl.ds(0, EB)], ibufs[b].at[1],
                              sems[b]).wait()
        pltpu.async_copy(h_hbm.at[ibufs[b].at[0]], rbufs[b], sems[b])

    def start_scatter(b):
        pltpu.make_async_copy(h_hbm.at[ibufs[b].at[0]], rbufs[b],
                              sems[b]).wait()
        pltpu.async_copy(rbufs[b], acc.at[ibufs[b].at[1]], sems[b], add=True)

    def wait_scatter(b):
        pltpu.make_async_copy(rbufs[b], acc.at[ibufs[b].at[1]],
                              sems[b]).wait()

    start_idx(0, 0)
    start_gather(0)

    def body(p, _):
        start_scatter(0)
        start_idx(1, 2 * p + 1)
        start_gather(1)
        wait_scatter(0)
        start_scatter(1)
        start_idx(0, 2 * p + 2)
        start_gather(0)
        wait_scatter(1)
        return 0
    lax.fori_loop(0, NP, body, 0)

    start_scatter(0)
    start_idx(1, NCHUNK - 1)
    start_gather(1)
    wait_scatter(0)
    start_scatter(1)
    wait_scatter(1)

    base = wid * E_PER + NCHUNK * EB
    pltpu.sync_copy(src_hbm.at[pl.ds(base, ET)], it.at[0])
    pltpu.sync_copy(dst_hbm.at[pl.ds(base, ET)], it.at[1])
    pltpu.async_copy(h_hbm.at[it.at[0]], rt, s0).wait()
    pltpu.sync_copy(rt, acc.at[it.at[1]], add=True)
    plsc.subcore_barrier()
    _writeback(acc, out_hbm, cid, sid)


def _make_agg(d):
    return pl.kernel(
        functools.partial(_agg_body, d=d),
        out_type=jax.ShapeDtypeStruct((NC, N, d), jnp.float32),
        mesh=_mesh,
        scratch_types=(
            [pltpu.VMEM((2, EB), jnp.int32)] * 2
            + [pltpu.VMEM((2, ET), jnp.int32)]
            + [pltpu.VMEM((EB, d), jnp.float32)] * 2
            + [pltpu.VMEM((ET, d), jnp.float32)]
            + [pltpu.VMEM_SHARED((N, d), jnp.float32)]
            + [pltpu.SemaphoreType.DMA] * 2
        ),
    )


_agg_h = _make_agg(H)


R = 1000


def _scale_from_counts(cnt_ref):
    cnt = cnt_ref[0, :, :] + cnt_ref[1, :, :]
    return lax.rsqrt(jnp.maximum(cnt[:, 0:1], 1.0))


def _l1_body(x_ref, co_ref, w_ref, o_ref):
    s = _scale_from_counts(co_ref)
    o_ref[...] = jnp.dot(x_ref[...] * s, w_ref[...],
                         preferred_element_type=jnp.float32)


def _mid_body(agg_ref, ci_ref, co_ref, b_ref, w_ref, o_ref):
    a = agg_ref[0, :, :] + agg_ref[1, :, :]
    si = _scale_from_counts(ci_ref)
    t = jnp.maximum(a * si + b_ref[...], 0.0)
    so = _scale_from_counts(co_ref)
    o_ref[...] = jnp.dot(t * so, w_ref[...],
                         preferred_element_type=jnp.float32)


def _final_body(agg_ref, ci_ref, b_ref, o_ref):
    a = agg_ref[0, :, :C] + agg_ref[1, :, :C]
    si = _scale_from_counts(ci_ref)
    o_ref[...] = a * si + b_ref[...]


def _cnt_spec():
    return pl.BlockSpec((NC, R, H), lambda i: (0, i, 0))


def _l1(x, cnt_out, w):
    return pl.pallas_call(
        _l1_body,
        grid=(N // R,),
        in_specs=[pl.BlockSpec((R, D_IN), lambda i: (i, 0)),
                  _cnt_spec(),
                  pl.BlockSpec((D_IN, H), lambda i: (0, 0))],
        out_specs=pl.BlockSpec((R, H), lambda i: (i, 0)),
        out_shape=jax.ShapeDtypeStruct((N, H), jnp.float32),
    )(x, cnt_out, w)


def _mid(agg, cnt_in, cnt_out, b, w, dout):
    return pl.pallas_call(
        _mid_body,
        grid=(N // R,),
        in_specs=[pl.BlockSpec((NC, R, H), lambda i: (0, i, 0)),
                  _cnt_spec(), _cnt_spec(),
                  pl.BlockSpec((1, H), lambda i: (0, 0)),
                  pl.BlockSpec((H, dout), lambda i: (0, 0))],
        out_specs=pl.BlockSpec((R, dout), lambda i: (i, 0)),
        out_shape=jax.ShapeDtypeStruct((N, dout), jnp.float32),
    )(agg, cnt_in, cnt_out, b.reshape(1, H), w)


def _final(agg, cnt_in, b):
    return pl.pallas_call(
        _final_body,
        grid=(N // R,),
        in_specs=[pl.BlockSpec((NC, R, H), lambda i: (0, i, 0)),
                  _cnt_spec(),
                  pl.BlockSpec((1, C), lambda i: (0, 0))],
        out_specs=pl.BlockSpec((R, C), lambda i: (i, 0)),
        out_shape=jax.ShapeDtypeStruct((N, C), jnp.float32),
    )(agg, cnt_in, b.reshape(1, C))


def kernel(features, edge_index, W1, b1, W2, b2, W3, b3):
    src = edge_index[0]
    dst = edge_index[1]
    cnt_out, cnt_in = _deg_kernel(src, dst)
    h1 = _l1(features, cnt_out, W1)
    agg1 = _agg_h(h1, src, dst)
    h2 = _mid(agg1, cnt_in, cnt_out, b1, W2, H)
    agg2 = _agg_h(h2, src, dst)
    W3p = jnp.pad(W3, ((0, 0), (0, H - C)))
    h3 = _mid(agg2, cnt_in, cnt_out, b2, W3p, H)
    agg3 = _agg_h(h3, src, dst)
    return _final(agg3, cnt_in, b3)


# speedup vs baseline: 1.1718x; 1.1718x over previous
import functools

import jax
import jax.numpy as jnp
from jax import lax
from jax.experimental import pallas as pl
from jax.experimental.pallas import tpu as pltpu
from jax.experimental.pallas import tpu_sc as plsc

N = 10000
E = 320000
D_IN = 128
H = 128
C = 64

NC = 2
NS = 16
NW = NC * NS
E_PER = E // NW
EB = 80
NCHUNK = E_PER // EB
RB = 624
REM = N - RB * NS
ZR = 208

_mesh = plsc.VectorSubcoreMesh(core_axis_name="c", subcore_axis_name="s")


def _fill_zeros(ref, rows, width):
    def body(i, _):
        def inner(j, _):
            ref[i, pl.ds(j * 16, 16)] = jnp.zeros((16,), jnp.float32)
            return 0
        return lax.fori_loop(0, width // 16, inner, 0)
    lax.fori_loop(0, rows, body, 0)


def _zero_acc(zero_v, acc, sid, width_rows=ZR):
    r0 = sid * RB
    for k in range(RB // ZR):
        pltpu.sync_copy(zero_v, acc.at[pl.ds(r0 + k * ZR, ZR)])

    @pl.when(sid == NS - 1)
    def _zero_tail():
        pltpu.sync_copy(zero_v.at[pl.ds(0, REM)], acc.at[pl.ds(RB * NS, REM)])


def _writeback(acc, out_hbm, cid, sid):
    r0 = sid * RB
    pltpu.sync_copy(acc.at[pl.ds(r0, RB)], out_hbm.at[cid, pl.ds(r0, RB)])

    @pl.when(sid == NS - 1)
    def _write_tail():
        pltpu.sync_copy(acc.at[pl.ds(RB * NS, REM)],
                        out_hbm.at[cid, pl.ds(RB * NS, REM)])


DEGW = 16
NPAD = 10240
NPT = NPAD // NS


def _deg_body(src_hbm, dst_hbm, out_hbm, in_hbm, ia, ib, hs, hd, rbuf, red,
              rep, sh_s, sh_d, sa, sb):
    cid = lax.axis_index("c")
    sid = lax.axis_index("s")
    wid = cid * NS + sid

    def zero(i, _):
        hs[pl.ds(i * 16, 16)] = jnp.zeros((16,), jnp.float32)
        hd[pl.ds(i * 16, 16)] = jnp.zeros((16,), jnp.float32)
        return 0
    lax.fori_loop(0, NPAD // 16, zero, 0)

    ones16 = jnp.ones((16,), jnp.float32)

    def load(buf, sem, g):
        base = wid * E_PER + g * EB
        pltpu.async_copy(src_hbm.at[pl.ds(base, EB)], buf.at[pl.ds(0, EB)],
                         sem)
        pltpu.async_copy(dst_hbm.at[pl.ds(base, EB)], buf.at[pl.ds(EB, EB)],
                         sem)

    def wait(buf, sem):
        pltpu.make_async_copy(src_hbm.at[pl.ds(0, EB)], buf.at[pl.ds(0, EB)],
                              sem).wait()
        pltpu.make_async_copy(src_hbm.at[pl.ds(0, EB)], buf.at[pl.ds(EB, EB)],
                              sem).wait()

    def scat(buf):
        for k in range(2 * EB // 16):
            v16 = buf[pl.ds(k * 16, 16)]
            plsc.addupdate_scatter(hs if k < EB // 16 else hd, [v16], ones16)

    load(ia, sa, 0)

    def body(p, _):
        wait(ia, sa)
        load(ib, sb, 2 * p + 1)
        scat(ia)
        wait(ib, sb)
        load(ia, sa, 2 * p + 2)
        scat(ib)
        return 0
    lax.fori_loop(0, (NCHUNK - 1) // 2, body, 0)
    wait(ia, sa)
    scat(ia)

    pltpu.sync_copy(hs, sh_s.at[pl.ds(sid * NPAD, NPAD)])
    pltpu.sync_copy(hd, sh_d.at[pl.ds(sid * NPAD, NPAD)])
    plsc.subcore_barrier()

    for shx, o_hbm in ((sh_s, out_hbm), (sh_d, in_hbm)):
        for r in range(NS):
            pltpu.sync_copy(shx.at[pl.ds(r * NPAD + sid * NPT, NPT)],
                            rbuf.at[pl.ds(r * NPT, NPT)])

        def reduce(j, _):
            acc16 = rbuf[pl.ds(j * 16, 16)]
            for r in range(1, NS):
                acc16 = acc16 + rbuf[pl.ds(r * NPT + j * 16, 16)]
            red[pl.ds(j * 16, 16)] = acc16
            return 0
        lax.fori_loop(0, NPT // 16, reduce, 0)

        def repl(j, _):
            c16 = red[pl.ds(j * 16, 16)]
            for t in range(16):
                rep[pl.ds(j * 256 + t * 16, 16)] = jnp.full(
                    (16,), c16[t], jnp.float32)
            return 0
        lax.fori_loop(0, NPT // 16, repl, 0)

        @pl.when(sid < NS - 1)
        def _full_wb():
            pltpu.sync_copy(rep, o_hbm.at[cid,
                                          pl.ds(sid * NPT * DEGW,
                                                NPT * DEGW)])

        @pl.when(sid == NS - 1)
        def _clip_wb():
            nlast = (N - (NS - 1) * NPT) * DEGW
            pltpu.sync_copy(rep.at[pl.ds(0, nlast)],
                            o_hbm.at[cid, pl.ds((NS - 1) * NPT * DEGW,
                                                nlast)])


_deg_kernel = pl.kernel(
    _deg_body,
    out_type=[jax.ShapeDtypeStruct((NC, N * DEGW), jnp.float32),
              jax.ShapeDtypeStruct((NC, N * DEGW), jnp.float32)],
    mesh=_mesh,
    compiler_params=pltpu.CompilerParams(needs_layout_passes=False),
    scratch_types=(
        [pltpu.VMEM((2 * EB,), jnp.int32)] * 2
        + [pltpu.VMEM((NPAD,), jnp.float32)] * 2
        + [pltpu.VMEM((NS * NPT,), jnp.float32),
           pltpu.VMEM((NPT,), jnp.float32),
           pltpu.VMEM((NPT * DEGW,), jnp.float32),
           pltpu.VMEM_SHARED((NS * NPAD,), jnp.float32),
           pltpu.VMEM_SHARED((NS * NPAD,), jnp.float32)]
        + [pltpu.SemaphoreType.DMA] * 2
    ),
)


def _agg_body(h_hbm, src_hbm, dst_hbm, out_hbm, i0, i1, i2, i3, r0, r1, r2,
              r3, acc, s0, s1, s2, s3, *, d):
    cid = lax.axis_index("c")
    sid = lax.axis_index("s")
    wid = cid * NS + sid
    ibufs = (i0, i1, i2, i3)
    rbufs = (r0, r1, r2, r3)
    sems = (s0, s1, s2, s3)

    _fill_zeros(r0, EB, d)
    zb = sid * RB
    for k in range(RB // EB):
        pltpu.sync_copy(r0, acc.at[pl.ds(zb + k * EB, EB)])
    pltpu.sync_copy(r0.at[pl.ds(0, RB % EB)],
                    acc.at[pl.ds(zb + RB - RB % EB, RB % EB)])

    @pl.when(sid == NS - 1)
    def _zero_tail():
        pltpu.sync_copy(r0.at[pl.ds(0, REM)], acc.at[pl.ds(RB * NS, REM)])
    plsc.subcore_barrier()

    SA, SB = (0, 1), (2, 3)

    def start_idx(pair, g):
        for j, b in enumerate(pair):
            base = wid * E_PER + (g * 2 + j) * EB
            pltpu.async_copy(src_hbm.at[pl.ds(base, EB)], ibufs[b].at[0],
                             sems[b])
            pltpu.async_copy(dst_hbm.at[pl.ds(base, EB)], ibufs[b].at[1],
                             sems[b])

    def start_gather(pair):
        for b in pair:
            pltpu.make_async_copy(src_hbm.at[pl.ds(0, EB)], ibufs[b].at[0],
                                  sems[b]).wait()
            pltpu.make_async_copy(src_hbm.at[pl.ds(0, EB)], ibufs[b].at[1],
                                  sems[b]).wait()
            pltpu.async_copy(h_hbm.at[ibufs[b].at[0]], rbufs[b], sems[b])

    def start_scatter(pair):
        for b in pair:
            pltpu.make_async_copy(h_hbm.at[ibufs[b].at[0]], rbufs[b],
                                  sems[b]).wait()
            pltpu.async_copy(rbufs[b], acc.at[ibufs[b].at[1]], sems[b],
                             add=True)

    def wait_scatter(pair):
        for b in pair:
            pltpu.make_async_copy(rbufs[b], acc.at[ibufs[b].at[1]],
                                  sems[b]).wait()

    NG = NCHUNK // 2
    NP = NG // 2 - 1

    start_idx(SA, 0)
    start_gather(SA)

    def body(p, _):
        start_scatter(SA)
        start_idx(SB, 2 * p + 1)
        start_gather(SB)
        wait_scatter(SA)
        start_scatter(SB)
        start_idx(SA, 2 * p + 2)
        start_gather(SA)
        wait_scatter(SB)
        return 0
    lax.fori_loop(0, NP, body, 0)

    start_scatter(SA)
    start_idx(SB, NG - 1)
    start_gather(SB)
    wait_scatter(SA)
    start_scatter(SB)
    wait_scatter(SB)
    for i in range(NG * 2, NCHUNK):
        base = wid * E_PER + i * EB
        pltpu.sync_copy(src_hbm.at[pl.ds(base, EB)], i0.at[0])
        pltpu.sync_copy(dst_hbm.at[pl.ds(base, EB)], i0.at[1])
        pltpu.async_copy(h_hbm.at[i0.at[0]], r0, s0).wait()
        pltpu.sync_copy(r0, acc.at[i0.at[1]], add=True)
    plsc.subcore_barrier()
    _writeback(acc, out_hbm, cid, sid)


def _make_agg(d):
    return pl.kernel(
        functools.partial(_agg_body, d=d),
        out_type=jax.ShapeDtypeStruct((NC, N, d), jnp.float32),
        mesh=_mesh,
        scratch_types=(
            [pltpu.VMEM((2, EB), jnp.int32)] * 4
            + [pltpu.VMEM((EB, d), jnp.float32)] * 4
            + [pltpu.VMEM_SHARED((N, d), jnp.float32)]
            + [pltpu.SemaphoreType.DMA] * 4
        ),
    )


_agg_h = _make_agg(H)


R = 1000


def _scale_from_counts(cnt_ref):
    cnt = cnt_ref[0, :, :] + cnt_ref[1, :, :]
    return lax.rsqrt(jnp.maximum(cnt[:, 0:1], 1.0))


def _l1_body(x_ref, co_ref, w_ref, o_ref):
    s = _scale_from_counts(co_ref)
    o_ref[...] = jnp.dot(x_ref[...] * s, w_ref[...],
                         preferred_element_type=jnp.float32)


def _mid_body(agg_ref, ci_ref, co_ref, b_ref, w_ref, o_ref):
    a = agg_ref[0, :, :] + agg_ref[1, :, :]
    si = _scale_from_counts(ci_ref)
    t = jnp.maximum(a * si + b_ref[...], 0.0)
    so = _scale_from_counts(co_ref)
    o_ref[...] = jnp.dot(t * so, w_ref[...],
                         preferred_element_type=jnp.float32)


def _final_body(agg_ref, ci_ref, b_ref, o_ref):
    a = agg_ref[0, :, :C] + agg_ref[1, :, :C]
    si = _scale_from_counts(ci_ref)
    o_ref[...] = a * si + b_ref[...]


def _cnt_spec():
    return pl.BlockSpec((NC, R, DEGW), lambda i: (0, i, 0))


def _l1(x, cnt_out, w):
    return pl.pallas_call(
        _l1_body,
        grid=(N // R,),
        in_specs=[pl.BlockSpec((R, D_IN), lambda i: (i, 0)),
                  _cnt_spec(),
                  pl.BlockSpec((D_IN, H), lambda i: (0, 0))],
        out_specs=pl.BlockSpec((R, H), lambda i: (i, 0)),
        out_shape=jax.ShapeDtypeStruct((N, H), jnp.float32),
    )(x, cnt_out, w)


def _mid(agg, cnt_in, cnt_out, b, w, dout):
    return pl.pallas_call(
        _mid_body,
        grid=(N // R,),
        in_specs=[pl.BlockSpec((NC, R, H), lambda i: (0, i, 0)),
                  _cnt_spec(), _cnt_spec(),
                  pl.BlockSpec((1, H), lambda i: (0, 0)),
                  pl.BlockSpec((H, dout), lambda i: (0, 0))],
        out_specs=pl.BlockSpec((R, dout), lambda i: (i, 0)),
        out_shape=jax.ShapeDtypeStruct((N, dout), jnp.float32),
    )(agg, cnt_in, cnt_out, b.reshape(1, H), w)


def _final(agg, cnt_in, b):
    return pl.pallas_call(
        _final_body,
        grid=(N // R,),
        in_specs=[pl.BlockSpec((NC, R, H), lambda i: (0, i, 0)),
                  _cnt_spec(),
                  pl.BlockSpec((1, C), lambda i: (0, 0))],
        out_specs=pl.BlockSpec((R, C), lambda i: (i, 0)),
        out_shape=jax.ShapeDtypeStruct((N, C), jnp.float32),
    )(agg, cnt_in, b.reshape(1, C))


def kernel(features, edge_index, W1, b1, W2, b2, W3, b3):
    src = edge_index[0]
    dst = edge_index[1]
    cnt_out, cnt_in = _deg_kernel(src, dst)
    cnt_out = cnt_out.reshape(NC, N, DEGW)
    cnt_in = cnt_in.reshape(NC, N, DEGW)
    h1 = _l1(features, cnt_out, W1)
    agg1 = _agg_h(h1, src, dst)
    h2 = _mid(agg1, cnt_in, cnt_out, b1, W2, H)
    agg2 = _agg_h(h2, src, dst)
    W3p = jnp.pad(W3, ((0, 0), (0, H - C)))
    h3 = _mid(agg2, cnt_in, cnt_out, b2, W3p, H)
    agg3 = _agg_h(h3, src, dst)
    return _final(agg3, cnt_in, b3)
